# baseline (speedup 1.0000x reference)
import functools

import jax
import jax.numpy as jnp
from jax import lax
from jax.experimental import pallas as pl
from jax.experimental.pallas import tpu as pltpu
from jax.experimental.pallas import tpu_sc as plsc

N_TOK = 2048
D_MODEL = 768
D_FF = 3072
N_EXP = 8
TOP_K = 2

ROW_B = 256
FH = D_FF // 2
GMAX = (N_TOK * TOP_K) // ROW_B + N_EXP
PAD = GMAX * ROW_B

NC, NS, LANES = 2, 16, 16
NW = NC * NS

CH = 128
NCH = (N_TOK * TOP_K) // CH


def _gelu_tanh(x):
    c = 0.7978845608028654
    return 0.5 * x * (1.0 + jnp.tanh(c * (x + 0.044715 * x * x * x)))


def _router_body(x_ref, wg_ref, p1_ref, p2_ref, g1_ref, g2_ref,
                 eids_ref, nused_ref, loss_ref, a_ref, c_ref):
    xb = x_ref[...]
    logits = jnp.dot(xb, wg_ref[...], preferred_element_type=jnp.float32)
    m = jnp.max(logits, axis=1, keepdims=True)
    p = jnp.exp(logits - m)
    probs = p / jnp.sum(p, axis=1, keepdims=True)
    lane = lax.broadcasted_iota(jnp.int32, (N_TOK, N_EXP), 1)
    m1 = jnp.max(probs, axis=1, keepdims=True)
    i1 = jnp.min(jnp.where(probs == m1, lane, N_EXP), axis=1, keepdims=True)
    sel1 = lane == i1
    probs2 = jnp.where(sel1, -jnp.inf, probs)
    m2 = jnp.max(probs2, axis=1, keepdims=True)
    i2 = jnp.min(jnp.where(probs2 == m2, lane, N_EXP), axis=1, keepdims=True)
    sel2 = lane == i2
    denom = m1 + m2
    g1_ref[...] = jnp.broadcast_to(m1 / denom, (N_TOK, LANES))
    g2_ref[...] = jnp.broadcast_to(m2 / denom, (N_TOK, LANES))

    s1f = sel1.astype(jnp.float32)
    s2f = sel2.astype(jnp.float32)
    a_ref[0:N_TOK, :] = s1f
    a_ref[N_TOK:2 * N_TOK, :] = s2f

    r_io = lax.broadcasted_iota(jnp.int32, (CH, CH), 0)
    c_io = lax.broadcasted_iota(jnp.int32, (CH, CH), 1)
    ltri = (c_io < r_io).astype(jnp.float32)

    def chunk(k, carry):
        ak = a_ref[pl.ds(k * CH, CH), :]
        c_ref[pl.ds(k * CH, CH), :] = (
            jnp.dot(ltri, ak, preferred_element_type=jnp.float32) + carry)
        return carry + jnp.sum(ak, axis=0, keepdims=True)

    counts = lax.fori_loop(0, NCH, chunk, jnp.zeros((1, N_EXP), jnp.float32))

    cnt_i = counts.astype(jnp.int32)
    nblk = (cnt_i + (ROW_B - 1)) // ROW_B
    nblk_f = nblk.astype(jnp.float32)
    r8 = lax.broadcasted_iota(jnp.int32, (N_EXP, N_EXP), 0)
    c8 = lax.broadcasted_iota(jnp.int32, (N_EXP, N_EXP), 1)
    upp8 = (r8 < c8).astype(jnp.float32)
    offb = jnp.dot(nblk_f, upp8, preferred_element_type=jnp.float32)
    offp = offb * ROW_B

    rank1 = jnp.sum(s1f * c_ref[0:N_TOK, :], axis=1, keepdims=True)
    rank2 = jnp.sum(s2f * c_ref[N_TOK:2 * N_TOK, :], axis=1, keepdims=True)
    off1 = jnp.sum(s1f * offp, axis=1, keepdims=True)
    off2 = jnp.sum(s2f * offp, axis=1, keepdims=True)
    p1_ref[...] = (off1 + rank1).astype(jnp.int32)
    p2_ref[...] = (off2 + rank2).astype(jnp.int32)

    ends = offb + nblk_f
    gi = lax.broadcasted_iota(jnp.int32, (1, GMAX), 1).astype(jnp.float32)
    lane1 = lax.broadcasted_iota(jnp.int32, (1, N_EXP), 1)
    eid = jnp.zeros((1, GMAX), jnp.float32)
    for e in range(N_EXP):
        end_e = jnp.sum(jnp.where(lane1 == e, ends, 0.0))
        eid = eid + (gi >= end_e).astype(jnp.float32)
    eids_ref[...] = jnp.minimum(eid, N_EXP - 1).astype(jnp.int32)
    nused_ref[...] = jnp.sum(
        jnp.where(lane1 == N_EXP - 1, ends, 0.0)).astype(jnp.int32
                                                         ).reshape(1, 1)

    mean_p = jnp.sum(probs, axis=0, keepdims=True) / N_TOK
    frac = counts / N_TOK
    loss_ref[...] = (N_EXP * jnp.sum(frac * mean_p)).reshape(1, 1)


def _router(x, Wg):
    return pl.pallas_call(
        _router_body,
        out_shape=[
            jax.ShapeDtypeStruct((N_TOK, 1), jnp.int32),
            jax.ShapeDtypeStruct((N_TOK, 1), jnp.int32),
            jax.ShapeDtypeStruct((N_TOK, LANES), jnp.float32),
            jax.ShapeDtypeStruct((N_TOK, LANES), jnp.float32),
            jax.ShapeDtypeStruct((1, GMAX), jnp.int32),
            jax.ShapeDtypeStruct((1, 1), jnp.int32),
            jax.ShapeDtypeStruct((1, 1), jnp.float32),
        ],
        scratch_shapes=[
            pltpu.VMEM((N_TOK * TOP_K, N_EXP), jnp.float32),
            pltpu.VMEM((N_TOK * TOP_K, N_EXP), jnp.float32),
        ],
    )(x, Wg)


CPW = N_TOK // NW


def _dispatch_body(x_hbm, p1_hbm, p2_hbm, xs_hbm, p1v, p2v, rowsv,
                   sem, semr):
    cid = lax.axis_index("c")
    sid = lax.axis_index("s")
    wid = sid * NC + cid
    tbase = wid * CPW
    c1 = pltpu.async_copy(p1_hbm.at[pl.ds(tbase, CPW)], p1v, sem)
    c2 = pltpu.async_copy(p2_hbm.at[pl.ds(tbase, CPW)], p2v, sem)
    c3 = pltpu.async_copy(x_hbm.at[pl.ds(tbase, CPW)], rowsv, semr)
    c1.wait()
    c2.wait()
    c3.wait()
    s1 = pltpu.async_copy(rowsv, xs_hbm.at[p1v], sem)
    s2 = pltpu.async_copy(rowsv, xs_hbm.at[p2v], semr)
    s1.wait()
    s2.wait()


def _dispatch(x, p1, p2):
    mesh = plsc.VectorSubcoreMesh(core_axis_name="c", subcore_axis_name="s",
                                  num_cores=NC, num_subcores=NS)
    f = pl.kernel(
        _dispatch_body,
        out_type=jax.ShapeDtypeStruct((PAD, D_MODEL), jnp.float32),
        mesh=mesh,
        scratch_types=[
            pltpu.VMEM((CPW,), jnp.int32),
            pltpu.VMEM((CPW,), jnp.int32),
            pltpu.VMEM((CPW, D_MODEL), jnp.float32),
            pltpu.SemaphoreType.DMA,
            pltpu.SemaphoreType.DMA,
        ],
    )
    return f(x, p1, p2)


def _grouped_body(eids_sref, nused_sref, xs_ref, w1_ref, b1_ref,
                  w2_ref, b2_ref, ys_ref):
    g = pl.program_id(0)
    f = pl.program_id(1)

    @pl.when(g < nused_sref[0])
    def _compute():
        h = jnp.dot(xs_ref[...], w1_ref[0],
                    preferred_element_type=jnp.float32,
                    precision=lax.Precision.DEFAULT)
        h = _gelu_tanh(h + b1_ref[0])
        y = jnp.dot(h, w2_ref[0],
                    preferred_element_type=jnp.float32,
                    precision=lax.Precision.DEFAULT)

        @pl.when(f == 0)
        def _first():
            ys_ref[...] = y + b2_ref[0]

        @pl.when(f > 0)
        def _acc():
            ys_ref[...] += y


def _grouped(eids, nused, xs, W1b, b1r, W2b, b2r):
    grid_spec = pltpu.PrefetchScalarGridSpec(
        num_scalar_prefetch=2,
        grid=(GMAX, D_FF // FH),
        in_specs=[
            pl.BlockSpec((ROW_B, D_MODEL), lambda g, f, E, nu: (g, 0)),
            pl.BlockSpec((1, D_MODEL, FH), lambda g, f, E, nu: (E[g], 0, f)),
            pl.BlockSpec((1, 1, FH), lambda g, f, E, nu: (E[g], 0, f)),
            pl.BlockSpec((1, FH, D_MODEL), lambda g, f, E, nu: (E[g], f, 0)),
            pl.BlockSpec((1, 1, D_MODEL), lambda g, f, E, nu: (E[g], 0, 0)),
        ],
        out_specs=pl.BlockSpec((ROW_B, D_MODEL), lambda g, f, E, nu: (g, 0)),
    )
    return pl.pallas_call(
        _grouped_body,
        grid_spec=grid_spec,
        out_shape=jax.ShapeDtypeStruct((PAD, D_MODEL), jnp.float32),
        compiler_params=pltpu.CompilerParams(
            dimension_semantics=("arbitrary", "arbitrary")),
    )(eids, nused, xs, W1b, b1r, W2b, b2r)


def _combine_body(ys_hbm, p1_hbm, p2_hbm, g1_hbm, g2_hbm, out_hbm,
                  p1v, p2v, g1v, g2v, av, bv, sem, semb):
    cid = lax.axis_index("c")
    sid = lax.axis_index("s")
    wid = sid * NC + cid
    tbase = wid * CPW
    c1 = pltpu.async_copy(p1_hbm.at[pl.ds(tbase, CPW)], p1v, sem)
    c2 = pltpu.async_copy(p2_hbm.at[pl.ds(tbase, CPW)], p2v, sem)
    c3 = pltpu.async_copy(g1_hbm.at[pl.ds(tbase, CPW)], g1v, semb)
    c4 = pltpu.async_copy(g2_hbm.at[pl.ds(tbase, CPW)], g2v, semb)
    c1.wait()
    c2.wait()
    ca = pltpu.async_copy(ys_hbm.at[p1v], av, sem)
    cb = pltpu.async_copy(ys_hbm.at[p2v], bv, semb)
    c3.wait()
    c4.wait()
    ca.wait()
    cb.wait()

    def row(r, _):
        ga = g1v[r, :]
        gb = g2v[r, :]
        for c in range(D_MODEL // LANES):
            av[r, pl.ds(c * LANES, LANES)] = (
                ga * av[r, pl.ds(c * LANES, LANES)]
                + gb * bv[r, pl.ds(c * LANES, LANES)])
        return 0

    lax.fori_loop(0, CPW, row, 0)
    pltpu.sync_copy(av, out_hbm.at[pl.ds(tbase, CPW)])


def _combine(ys, p1, p2, g1e, g2e):
    mesh = plsc.VectorSubcoreMesh(core_axis_name="c", subcore_axis_name="s",
                                  num_cores=NC, num_subcores=NS)
    f = pl.kernel(
        _combine_body,
        out_type=jax.ShapeDtypeStruct((N_TOK, D_MODEL), jnp.float32),
        mesh=mesh,
        scratch_types=[
            pltpu.VMEM((CPW,), jnp.int32),
            pltpu.VMEM((CPW,), jnp.int32),
            pltpu.VMEM((CPW, LANES), jnp.float32),
            pltpu.VMEM((CPW, LANES), jnp.float32),
            pltpu.VMEM((CPW, D_MODEL), jnp.float32),
            pltpu.VMEM((CPW, D_MODEL), jnp.float32),
            pltpu.SemaphoreType.DMA,
            pltpu.SemaphoreType.DMA,
        ],
    )
    return f(ys, p1, p2, g1e, g2e)


def kernel(x, Wg, W1, b1, W2, b2):
    p1, p2, g1e, g2e, eids, nused, loss = _router(x, Wg)
    p1 = p1.reshape(N_TOK)
    p2 = p2.reshape(N_TOK)
    xs = _dispatch(x, p1, p2)
    ys = _grouped(eids.reshape(GMAX), nused.reshape(1), xs,
                  W1, b1.reshape(N_EXP, 1, D_FF),
                  W2, b2.reshape(N_EXP, 1, D_MODEL))
    out = _combine(ys, p1, p2, g1e, g2e)
    return out, loss.reshape(())


# speedup vs baseline: 1.3006x; 1.3006x over previous
import functools

import jax
import jax.numpy as jnp
from jax import lax
from jax.experimental import pallas as pl
from jax.experimental.pallas import tpu as pltpu
from jax.experimental.pallas import tpu_sc as plsc

N_TOK = 2048
D_MODEL = 768
D_FF = 3072
N_EXP = 8
TOP_K = 2

ROW_B = 256
FH = D_FF
GMAX = (N_TOK * TOP_K) // ROW_B + N_EXP
PAD = GMAX * ROW_B

NC, NS, LANES = 2, 16, 16
NW = NC * NS

CH = 128
NCH = (N_TOK * TOP_K) // CH


def _gelu_tanh(x):
    c = 0.7978845608028654
    return 0.5 * x * (1.0 + jnp.tanh(c * (x + 0.044715 * x * x * x)))


def _router_body(x_ref, wg_ref, p1_ref, p2_ref, g1_ref, g2_ref,
                 eids_ref, nused_ref, loss_ref, a_ref, c_ref):
    xb = x_ref[...]
    logits = jnp.dot(xb, wg_ref[...], preferred_element_type=jnp.float32)
    m = jnp.max(logits, axis=1, keepdims=True)
    p = jnp.exp(logits - m)
    probs = p / jnp.sum(p, axis=1, keepdims=True)
    lane = lax.broadcasted_iota(jnp.int32, (N_TOK, N_EXP), 1)
    m1 = jnp.max(probs, axis=1, keepdims=True)
    i1 = jnp.min(jnp.where(probs == m1, lane, N_EXP), axis=1, keepdims=True)
    sel1 = lane == i1
    probs2 = jnp.where(sel1, -jnp.inf, probs)
    m2 = jnp.max(probs2, axis=1, keepdims=True)
    i2 = jnp.min(jnp.where(probs2 == m2, lane, N_EXP), axis=1, keepdims=True)
    sel2 = lane == i2
    denom = m1 + m2
    g1_ref[...] = jnp.broadcast_to(m1 / denom, (N_TOK, LANES))
    g2_ref[...] = jnp.broadcast_to(m2 / denom, (N_TOK, LANES))

    s1f = sel1.astype(jnp.float32)
    s2f = sel2.astype(jnp.float32)
    a_ref[0:N_TOK, :] = s1f
    a_ref[N_TOK:2 * N_TOK, :] = s2f

    r_io = lax.broadcasted_iota(jnp.int32, (CH, CH), 0)
    c_io = lax.broadcasted_iota(jnp.int32, (CH, CH), 1)
    ltri = (c_io < r_io).astype(jnp.float32)

    def chunk(k, carry):
        ak = a_ref[pl.ds(k * CH, CH), :]
        c_ref[pl.ds(k * CH, CH), :] = (
            jnp.dot(ltri, ak, preferred_element_type=jnp.float32) + carry)
        return carry + jnp.sum(ak, axis=0, keepdims=True)

    counts = lax.fori_loop(0, NCH, chunk, jnp.zeros((1, N_EXP), jnp.float32))

    cnt_i = counts.astype(jnp.int32)
    nblk = (cnt_i + (ROW_B - 1)) // ROW_B
    nblk_f = nblk.astype(jnp.float32)
    r8 = lax.broadcasted_iota(jnp.int32, (N_EXP, N_EXP), 0)
    c8 = lax.broadcasted_iota(jnp.int32, (N_EXP, N_EXP), 1)
    upp8 = (r8 < c8).astype(jnp.float32)
    offb = jnp.dot(nblk_f, upp8, preferred_element_type=jnp.float32)
    offp = offb * ROW_B

    rank1 = jnp.sum(s1f * c_ref[0:N_TOK, :], axis=1, keepdims=True)
    rank2 = jnp.sum(s2f * c_ref[N_TOK:2 * N_TOK, :], axis=1, keepdims=True)
    off1 = jnp.sum(s1f * offp, axis=1, keepdims=True)
    off2 = jnp.sum(s2f * offp, axis=1, keepdims=True)
    p1_ref[...] = (off1 + rank1).astype(jnp.int32)
    p2_ref[...] = (off2 + rank2).astype(jnp.int32)

    ends = offb + nblk_f
    gi = lax.broadcasted_iota(jnp.int32, (1, GMAX), 1).astype(jnp.float32)
    lane1 = lax.broadcasted_iota(jnp.int32, (1, N_EXP), 1)
    eid = jnp.zeros((1, GMAX), jnp.float32)
    for e in range(N_EXP):
        end_e = jnp.sum(jnp.where(lane1 == e, ends, 0.0))
        eid = eid + (gi >= end_e).astype(jnp.float32)
    eids_ref[...] = jnp.minimum(eid, N_EXP - 1).astype(jnp.int32)
    nused_ref[...] = jnp.sum(
        jnp.where(lane1 == N_EXP - 1, ends, 0.0)).astype(jnp.int32
                                                         ).reshape(1, 1)

    mean_p = jnp.sum(probs, axis=0, keepdims=True) / N_TOK
    frac = counts / N_TOK
    loss_ref[...] = (N_EXP * jnp.sum(frac * mean_p)).reshape(1, 1)


def _router(x, Wg):
    return pl.pallas_call(
        _router_body,
        out_shape=[
            jax.ShapeDtypeStruct((N_TOK, 1), jnp.int32),
            jax.ShapeDtypeStruct((N_TOK, 1), jnp.int32),
            jax.ShapeDtypeStruct((N_TOK, LANES), jnp.float32),
            jax.ShapeDtypeStruct((N_TOK, LANES), jnp.float32),
            jax.ShapeDtypeStruct((1, GMAX), jnp.int32),
            jax.ShapeDtypeStruct((1, 1), jnp.int32),
            jax.ShapeDtypeStruct((1, 1), jnp.float32),
        ],
        scratch_shapes=[
            pltpu.VMEM((N_TOK * TOP_K, N_EXP), jnp.float32),
            pltpu.VMEM((N_TOK * TOP_K, N_EXP), jnp.float32),
        ],
    )(x, Wg)


CPW = N_TOK // NW


def _dispatch_body(x_hbm, p1_hbm, p2_hbm, xs_hbm, p1v, p2v, rowsv,
                   sem, semr):
    cid = lax.axis_index("c")
    sid = lax.axis_index("s")
    wid = sid * NC + cid
    tbase = wid * CPW
    c1 = pltpu.async_copy(p1_hbm.at[pl.ds(tbase, CPW)], p1v, sem)
    c2 = pltpu.async_copy(p2_hbm.at[pl.ds(tbase, CPW)], p2v, sem)
    c3 = pltpu.async_copy(x_hbm.at[pl.ds(tbase, CPW)], rowsv, semr)
    c1.wait()
    c2.wait()
    c3.wait()
    s1 = pltpu.async_copy(rowsv, xs_hbm.at[p1v], sem)
    s2 = pltpu.async_copy(rowsv, xs_hbm.at[p2v], semr)
    s1.wait()
    s2.wait()


def _dispatch(x, p1, p2):
    mesh = plsc.VectorSubcoreMesh(core_axis_name="c", subcore_axis_name="s",
                                  num_cores=NC, num_subcores=NS)
    f = pl.kernel(
        _dispatch_body,
        out_type=jax.ShapeDtypeStruct((PAD, D_MODEL), jnp.float32),
        mesh=mesh,
        scratch_types=[
            pltpu.VMEM((CPW,), jnp.int32),
            pltpu.VMEM((CPW,), jnp.int32),
            pltpu.VMEM((CPW, D_MODEL), jnp.float32),
            pltpu.SemaphoreType.DMA,
            pltpu.SemaphoreType.DMA,
        ],
    )
    return f(x, p1, p2)


def _grouped_body(eids_sref, nused_sref, xs_ref, w1_ref, b1_ref,
                  w2_ref, b2_ref, ys_ref):
    g = pl.program_id(0)
    f = pl.program_id(1)

    @pl.when(g < nused_sref[0])
    def _compute():
        h = jnp.dot(xs_ref[...], w1_ref[0],
                    preferred_element_type=jnp.float32,
                    precision=lax.Precision.DEFAULT)
        h = _gelu_tanh(h + b1_ref[0])
        y = jnp.dot(h, w2_ref[0],
                    preferred_element_type=jnp.float32,
                    precision=lax.Precision.DEFAULT)

        @pl.when(f == 0)
        def _first():
            ys_ref[...] = y + b2_ref[0]

        @pl.when(f > 0)
        def _acc():
            ys_ref[...] += y


def _grouped(eids, nused, xs, W1b, b1r, W2b, b2r):
    grid_spec = pltpu.PrefetchScalarGridSpec(
        num_scalar_prefetch=2,
        grid=(GMAX, D_FF // FH),
        in_specs=[
            pl.BlockSpec((ROW_B, D_MODEL), lambda g, f, E, nu: (g, 0)),
            pl.BlockSpec((1, D_MODEL, FH), lambda g, f, E, nu: (E[g], 0, f)),
            pl.BlockSpec((1, 1, FH), lambda g, f, E, nu: (E[g], 0, f)),
            pl.BlockSpec((1, FH, D_MODEL), lambda g, f, E, nu: (E[g], f, 0)),
            pl.BlockSpec((1, 1, D_MODEL), lambda g, f, E, nu: (E[g], 0, 0)),
        ],
        out_specs=pl.BlockSpec((ROW_B, D_MODEL), lambda g, f, E, nu: (g, 0)),
    )
    return pl.pallas_call(
        _grouped_body,
        grid_spec=grid_spec,
        out_shape=jax.ShapeDtypeStruct((PAD, D_MODEL), jnp.float32),
        compiler_params=pltpu.CompilerParams(
            dimension_semantics=("arbitrary", "arbitrary")),
    )(eids, nused, xs, W1b, b1r, W2b, b2r)


def _combine_body(ys_hbm, p1_hbm, p2_hbm, g1_hbm, g2_hbm, out_hbm,
                  p1v, p2v, g1v, g2v, av, bv, sem, semb):
    cid = lax.axis_index("c")
    sid = lax.axis_index("s")
    wid = sid * NC + cid
    tbase = wid * CPW
    c1 = pltpu.async_copy(p1_hbm.at[pl.ds(tbase, CPW)], p1v, sem)
    c2 = pltpu.async_copy(p2_hbm.at[pl.ds(tbase, CPW)], p2v, sem)
    c3 = pltpu.async_copy(g1_hbm.at[pl.ds(tbase, CPW)], g1v, semb)
    c4 = pltpu.async_copy(g2_hbm.at[pl.ds(tbase, CPW)], g2v, semb)
    c1.wait()
    c2.wait()
    ca = pltpu.async_copy(ys_hbm.at[p1v], av, sem)
    cb = pltpu.async_copy(ys_hbm.at[p2v], bv, semb)
    c3.wait()
    c4.wait()
    ca.wait()
    cb.wait()

    def row(r, _):
        ga = g1v[r, :]
        gb = g2v[r, :]
        for c in range(D_MODEL // LANES):
            av[r, pl.ds(c * LANES, LANES)] = (
                ga * av[r, pl.ds(c * LANES, LANES)]
                + gb * bv[r, pl.ds(c * LANES, LANES)])
        return 0

    lax.fori_loop(0, CPW, row, 0)
    pltpu.sync_copy(av, out_hbm.at[pl.ds(tbase, CPW)])


def _combine(ys, p1, p2, g1e, g2e):
    mesh = plsc.VectorSubcoreMesh(core_axis_name="c", subcore_axis_name="s",
                                  num_cores=NC, num_subcores=NS)
    f = pl.kernel(
        _combine_body,
        out_type=jax.ShapeDtypeStruct((N_TOK, D_MODEL), jnp.float32),
        mesh=mesh,
        scratch_types=[
            pltpu.VMEM((CPW,), jnp.int32),
            pltpu.VMEM((CPW,), jnp.int32),
            pltpu.VMEM((CPW, LANES), jnp.float32),
            pltpu.VMEM((CPW, LANES), jnp.float32),
            pltpu.VMEM((CPW, D_MODEL), jnp.float32),
            pltpu.VMEM((CPW, D_MODEL), jnp.float32),
            pltpu.SemaphoreType.DMA,
            pltpu.SemaphoreType.DMA,
        ],
    )
    return f(ys, p1, p2, g1e, g2e)


def kernel(x, Wg, W1, b1, W2, b2):
    p1, p2, g1e, g2e, eids, nused, loss = _router(x, Wg)
    p1 = p1.reshape(N_TOK)
    p2 = p2.reshape(N_TOK)
    xs = _dispatch(x, p1, p2)
    ys = _grouped(eids.reshape(GMAX), nused.reshape(1), xs,
                  W1, b1.reshape(N_EXP, 1, D_FF),
                  W2, b2.reshape(N_EXP, 1, D_MODEL))
    out = _combine(ys, p1, p2, g1e, g2e)
    return out, loss.reshape(())


# speedup vs baseline: 1.4044x; 1.0798x over previous
import functools

import jax
import jax.numpy as jnp
from jax import lax
from jax.experimental import pallas as pl
from jax.experimental.pallas import tpu as pltpu
from jax.experimental.pallas import tpu_sc as plsc

N_TOK = 2048
D_MODEL = 768
D_FF = 3072
N_EXP = 8
TOP_K = 2

ROW_B = 512
FH = D_FF
GMAX = (N_TOK * TOP_K) // ROW_B + N_EXP
PAD = GMAX * ROW_B

NC, NS, LANES = 2, 16, 16
NW = NC * NS

CH = 128
NCH = (N_TOK * TOP_K) // CH


def _gelu_tanh(x):
    c = 0.7978845608028654
    return 0.5 * x * (1.0 + jnp.tanh(c * (x + 0.044715 * x * x * x)))


def _router_body(x_ref, wg_ref, p1_ref, p2_ref, g1_ref, g2_ref,
                 eids_ref, nused_ref, loss_ref, a_ref, c_ref):
    xb = x_ref[...]
    logits = jnp.dot(xb, wg_ref[...], preferred_element_type=jnp.float32)
    m = jnp.max(logits, axis=1, keepdims=True)
    p = jnp.exp(logits - m)
    probs = p / jnp.sum(p, axis=1, keepdims=True)
    lane = lax.broadcasted_iota(jnp.int32, (N_TOK, N_EXP), 1)
    m1 = jnp.max(probs, axis=1, keepdims=True)
    i1 = jnp.min(jnp.where(probs == m1, lane, N_EXP), axis=1, keepdims=True)
    sel1 = lane == i1
    probs2 = jnp.where(sel1, -jnp.inf, probs)
    m2 = jnp.max(probs2, axis=1, keepdims=True)
    i2 = jnp.min(jnp.where(probs2 == m2, lane, N_EXP), axis=1, keepdims=True)
    sel2 = lane == i2
    denom = m1 + m2
    g1_ref[...] = jnp.broadcast_to(m1 / denom, (N_TOK, LANES))
    g2_ref[...] = jnp.broadcast_to(m2 / denom, (N_TOK, LANES))

    s1f = sel1.astype(jnp.float32)
    s2f = sel2.astype(jnp.float32)
    a_ref[0:N_TOK, :] = s1f
    a_ref[N_TOK:2 * N_TOK, :] = s2f

    r_io = lax.broadcasted_iota(jnp.int32, (CH, CH), 0)
    c_io = lax.broadcasted_iota(jnp.int32, (CH, CH), 1)
    ltri = (c_io < r_io).astype(jnp.float32)

    def chunk(k, carry):
        ak = a_ref[pl.ds(k * CH, CH), :]
        c_ref[pl.ds(k * CH, CH), :] = (
            jnp.dot(ltri, ak, preferred_element_type=jnp.float32) + carry)
        return carry + jnp.sum(ak, axis=0, keepdims=True)

    counts = lax.fori_loop(0, NCH, chunk, jnp.zeros((1, N_EXP), jnp.float32))

    cnt_i = counts.astype(jnp.int32)
    nblk = (cnt_i + (ROW_B - 1)) // ROW_B
    nblk_f = nblk.astype(jnp.float32)
    r8 = lax.broadcasted_iota(jnp.int32, (N_EXP, N_EXP), 0)
    c8 = lax.broadcasted_iota(jnp.int32, (N_EXP, N_EXP), 1)
    upp8 = (r8 < c8).astype(jnp.float32)
    offb = jnp.dot(nblk_f, upp8, preferred_element_type=jnp.float32)
    offp = offb * ROW_B

    rank1 = jnp.sum(s1f * c_ref[0:N_TOK, :], axis=1, keepdims=True)
    rank2 = jnp.sum(s2f * c_ref[N_TOK:2 * N_TOK, :], axis=1, keepdims=True)
    off1 = jnp.sum(s1f * offp, axis=1, keepdims=True)
    off2 = jnp.sum(s2f * offp, axis=1, keepdims=True)
    p1_ref[...] = (off1 + rank1).astype(jnp.int32)
    p2_ref[...] = (off2 + rank2).astype(jnp.int32)

    ends = offb + nblk_f
    gi = lax.broadcasted_iota(jnp.int32, (1, GMAX), 1).astype(jnp.float32)
    lane1 = lax.broadcasted_iota(jnp.int32, (1, N_EXP), 1)
    eid = jnp.zeros((1, GMAX), jnp.float32)
    for e in range(N_EXP):
        end_e = jnp.sum(jnp.where(lane1 == e, ends, 0.0))
        eid = eid + (gi >= end_e).astype(jnp.float32)
    eids_ref[...] = jnp.minimum(eid, N_EXP - 1).astype(jnp.int32)
    nused_ref[...] = jnp.sum(
        jnp.where(lane1 == N_EXP - 1, ends, 0.0)).astype(jnp.int32
                                                         ).reshape(1, 1)

    mean_p = jnp.sum(probs, axis=0, keepdims=True) / N_TOK
    frac = counts / N_TOK
    loss_ref[...] = (N_EXP * jnp.sum(frac * mean_p)).reshape(1, 1)


def _router(x, Wg):
    return pl.pallas_call(
        _router_body,
        out_shape=[
            jax.ShapeDtypeStruct((N_TOK, 1), jnp.int32),
            jax.ShapeDtypeStruct((N_TOK, 1), jnp.int32),
            jax.ShapeDtypeStruct((N_TOK, LANES), jnp.float32),
            jax.ShapeDtypeStruct((N_TOK, LANES), jnp.float32),
            jax.ShapeDtypeStruct((1, GMAX), jnp.int32),
            jax.ShapeDtypeStruct((1, 1), jnp.int32),
            jax.ShapeDtypeStruct((1, 1), jnp.float32),
        ],
        scratch_shapes=[
            pltpu.VMEM((N_TOK * TOP_K, N_EXP), jnp.float32),
            pltpu.VMEM((N_TOK * TOP_K, N_EXP), jnp.float32),
        ],
    )(x, Wg)


CPW = N_TOK // NW


def _dispatch_body(x_hbm, p1_hbm, p2_hbm, xs_hbm, p1v, p2v, rowsv,
                   sem, semr):
    cid = lax.axis_index("c")
    sid = lax.axis_index("s")
    wid = sid * NC + cid
    tbase = wid * CPW
    c1 = pltpu.async_copy(p1_hbm.at[pl.ds(tbase, CPW)], p1v, sem)
    c2 = pltpu.async_copy(p2_hbm.at[pl.ds(tbase, CPW)], p2v, sem)
    c3 = pltpu.async_copy(x_hbm.at[pl.ds(tbase, CPW)], rowsv, semr)
    c1.wait()
    c2.wait()
    c3.wait()
    s1 = pltpu.async_copy(rowsv, xs_hbm.at[p1v], sem)
    s2 = pltpu.async_copy(rowsv, xs_hbm.at[p2v], semr)
    s1.wait()
    s2.wait()


def _dispatch(x, p1, p2):
    mesh = plsc.VectorSubcoreMesh(core_axis_name="c", subcore_axis_name="s",
                                  num_cores=NC, num_subcores=NS)
    f = pl.kernel(
        _dispatch_body,
        out_type=jax.ShapeDtypeStruct((PAD, D_MODEL), jnp.float32),
        mesh=mesh,
        scratch_types=[
            pltpu.VMEM((CPW,), jnp.int32),
            pltpu.VMEM((CPW,), jnp.int32),
            pltpu.VMEM((CPW, D_MODEL), jnp.float32),
            pltpu.SemaphoreType.DMA,
            pltpu.SemaphoreType.DMA,
        ],
    )
    return f(x, p1, p2)


def _grouped_body(eids_sref, nused_sref, xs_ref, w1_ref, b1_ref,
                  w2_ref, b2_ref, ys_ref):
    g = pl.program_id(0)
    f = pl.program_id(1)

    @pl.when(g < nused_sref[0])
    def _compute():
        h = jnp.dot(xs_ref[...], w1_ref[0],
                    preferred_element_type=jnp.float32,
                    precision=lax.Precision.DEFAULT)
        h = _gelu_tanh(h + b1_ref[0])
        y = jnp.dot(h, w2_ref[0],
                    preferred_element_type=jnp.float32,
                    precision=lax.Precision.DEFAULT)

        @pl.when(f == 0)
        def _first():
            ys_ref[...] = y + b2_ref[0]

        @pl.when(f > 0)
        def _acc():
            ys_ref[...] += y


def _grouped(eids, nused, xs, W1b, b1r, W2b, b2r):
    grid_spec = pltpu.PrefetchScalarGridSpec(
        num_scalar_prefetch=2,
        grid=(GMAX, D_FF // FH),
        in_specs=[
            pl.BlockSpec((ROW_B, D_MODEL), lambda g, f, E, nu: (g, 0)),
            pl.BlockSpec((1, D_MODEL, FH), lambda g, f, E, nu: (E[g], 0, f)),
            pl.BlockSpec((1, 1, FH), lambda g, f, E, nu: (E[g], 0, f)),
            pl.BlockSpec((1, FH, D_MODEL), lambda g, f, E, nu: (E[g], f, 0)),
            pl.BlockSpec((1, 1, D_MODEL), lambda g, f, E, nu: (E[g], 0, 0)),
        ],
        out_specs=pl.BlockSpec((ROW_B, D_MODEL), lambda g, f, E, nu: (g, 0)),
    )
    return pl.pallas_call(
        _grouped_body,
        grid_spec=grid_spec,
        out_shape=jax.ShapeDtypeStruct((PAD, D_MODEL), jnp.float32),
        compiler_params=pltpu.CompilerParams(
            dimension_semantics=("arbitrary", "arbitrary")),
    )(eids, nused, xs, W1b, b1r, W2b, b2r)


def _combine_body(ys_hbm, p1_hbm, p2_hbm, g1_hbm, g2_hbm, out_hbm,
                  p1v, p2v, g1v, g2v, av, bv, sem, semb):
    cid = lax.axis_index("c")
    sid = lax.axis_index("s")
    wid = sid * NC + cid
    tbase = wid * CPW
    c1 = pltpu.async_copy(p1_hbm.at[pl.ds(tbase, CPW)], p1v, sem)
    c2 = pltpu.async_copy(p2_hbm.at[pl.ds(tbase, CPW)], p2v, sem)
    c3 = pltpu.async_copy(g1_hbm.at[pl.ds(tbase, CPW)], g1v, semb)
    c4 = pltpu.async_copy(g2_hbm.at[pl.ds(tbase, CPW)], g2v, semb)
    c1.wait()
    c2.wait()
    ca = pltpu.async_copy(ys_hbm.at[p1v], av, sem)
    cb = pltpu.async_copy(ys_hbm.at[p2v], bv, semb)
    c3.wait()
    c4.wait()
    ca.wait()
    cb.wait()

    def row(r, _):
        ga = g1v[r, :]
        gb = g2v[r, :]
        for c in range(D_MODEL // LANES):
            av[r, pl.ds(c * LANES, LANES)] = (
                ga * av[r, pl.ds(c * LANES, LANES)]
                + gb * bv[r, pl.ds(c * LANES, LANES)])
        return 0

    lax.fori_loop(0, CPW, row, 0)
    pltpu.sync_copy(av, out_hbm.at[pl.ds(tbase, CPW)])


def _combine(ys, p1, p2, g1e, g2e):
    mesh = plsc.VectorSubcoreMesh(core_axis_name="c", subcore_axis_name="s",
                                  num_cores=NC, num_subcores=NS)
    f = pl.kernel(
        _combine_body,
        out_type=jax.ShapeDtypeStruct((N_TOK, D_MODEL), jnp.float32),
        mesh=mesh,
        scratch_types=[
            pltpu.VMEM((CPW,), jnp.int32),
            pltpu.VMEM((CPW,), jnp.int32),
            pltpu.VMEM((CPW, LANES), jnp.float32),
            pltpu.VMEM((CPW, LANES), jnp.float32),
            pltpu.VMEM((CPW, D_MODEL), jnp.float32),
            pltpu.VMEM((CPW, D_MODEL), jnp.float32),
            pltpu.SemaphoreType.DMA,
            pltpu.SemaphoreType.DMA,
        ],
    )
    return f(ys, p1, p2, g1e, g2e)


def kernel(x, Wg, W1, b1, W2, b2):
    p1, p2, g1e, g2e, eids, nused, loss = _router(x, Wg)
    p1 = p1.reshape(N_TOK)
    p2 = p2.reshape(N_TOK)
    xs = _dispatch(x, p1, p2)
    ys = _grouped(eids.reshape(GMAX), nused.reshape(1), xs,
                  W1, b1.reshape(N_EXP, 1, D_FF),
                  W2, b2.reshape(N_EXP, 1, D_MODEL))
    out = _combine(ys, p1, p2, g1e, g2e)
    return out, loss.reshape(())
